# baseline (speedup 1.0000x reference)
import jax
import jax.numpy as jnp
from jax import lax
from jax.experimental import pallas as pl
from jax.experimental.pallas import tpu as pltpu
from jax.experimental.pallas import tpu_sc as plsc

N_ROWS = 4096
N_K = 20
N_CLASSES = 1000
N_WORKERS = 32
ROWS_PER_W = N_ROWS // N_WORKERS
G = 4
STEPS = ROWS_PER_W // G
LABELS_PER_STEP = G * N_K
VECS = LABELS_PER_STEP // 16
LAB_PER_W = ROWS_PER_W * N_K


def _sc_body(labels_hbm, a_hbm, k_hbm, ones_hbm, zeros16_hbm, zerosblk_hbm,
             out_hbm, lab_v, a_v, k_v, ones_v, zer_v, buf, sem):
    wid = lax.axis_index("s") * 2 + lax.axis_index("c")
    lab_base = wid * LAB_PER_W
    pltpu.sync_copy(labels_hbm.at[pl.ds(lab_base, LAB_PER_W)], lab_v)
    pltpu.sync_copy(a_hbm.at[pl.ds(lab_base, LAB_PER_W)], a_v)
    pltpu.sync_copy(k_hbm.at[pl.ds(lab_base, LAB_PER_W)], k_v)
    pltpu.sync_copy(ones_hbm, ones_v)
    pltpu.sync_copy(zeros16_hbm, zer_v)
    pltpu.sync_copy(zerosblk_hbm, buf)

    for t in range(STEPS):
        for v in range(VECS):
            off = t * LABELS_PER_STEP + v * 16
            plsc.store_scatter(
                buf,
                [a_v[pl.ds(off, 16)], k_v[pl.ds(off, 16)],
                 lab_v[pl.ds(off, 16)]],
                ones_v[...],
            )
        row0 = wid * ROWS_PER_W + t * G
        pltpu.async_copy(buf, out_hbm.at[pl.ds(row0, G)], sem).wait()
        for v in range(VECS):
            off = t * LABELS_PER_STEP + v * 16
            plsc.store_scatter(
                buf,
                [a_v[pl.ds(off, 16)], k_v[pl.ds(off, 16)],
                 lab_v[pl.ds(off, 16)]],
                zer_v[...],
            )


_sc_onehot = pl.kernel(
    _sc_body,
    out_type=jax.ShapeDtypeStruct((N_ROWS, N_K, N_CLASSES), jnp.float32),
    mesh=plsc.VectorSubcoreMesh(core_axis_name="c", subcore_axis_name="s"),
    compiler_params=pltpu.CompilerParams(needs_layout_passes=False),
    scratch_types=[
        pltpu.VMEM((LAB_PER_W,), jnp.int32),
        pltpu.VMEM((LAB_PER_W,), jnp.int32),
        pltpu.VMEM((LAB_PER_W,), jnp.int32),
        pltpu.VMEM((16,), jnp.float32),
        pltpu.VMEM((16,), jnp.float32),
        pltpu.VMEM((G, N_K, N_CLASSES), jnp.float32),
        pltpu.SemaphoreType.DMA,
    ],
)


def kernel(labels):
    labels_flat = labels.reshape(N_ROWS * N_K)
    m = jnp.arange(N_ROWS * N_K, dtype=jnp.int32)
    a_idx = (m // N_K) % G
    k_idx = m % N_K
    ones16 = jnp.ones((16,), jnp.float32)
    zeros16 = jnp.zeros((16,), jnp.float32)
    zeros_blk = jnp.zeros((G, N_K, N_CLASSES), jnp.float32)
    return _sc_onehot(labels_flat, a_idx, k_idx, ones16, zeros16, zeros_blk)
